# baseline (speedup 1.0000x reference)
import functools

import jax
import jax.numpy as jnp
from jax import lax
from jax.experimental import pallas as pl
from jax.experimental.pallas import tpu as pltpu
from jax.experimental.pallas import tpu_sc as plsc

DIM = 64
CHUNK = 128
LANES = 16
NC, NS = 2, 16
NW = NC * NS
GROUPS = CHUNK // LANES
NBUF = 4


def _rsqrt(s):
    i = plsc.bitcast(s, jnp.int32)
    i = jnp.int32(0x5F3759DF) - lax.shift_right_logical(i, 1)
    y = plsc.bitcast(i, jnp.float32)
    for _ in range(3):
        y = y * (1.5 - 0.5 * s * y * y)
    return y


def _shuffle(x, idx):
    dn = lax.GatherDimensionNumbers(
        offset_dims=(), collapsed_slice_dims=(0,), start_index_map=(0,)
    )
    return lax.gather(
        x, idx[:, None], dn, (1,),
        mode=lax.GatherScatterMode.PROMISE_IN_BOUNDS,
    )


def _hsum(acc):
    lanes = lax.iota(jnp.int32, LANES)
    for sh in (8, 4, 2, 1):
        perm = jnp.bitwise_xor(lanes, sh)
        acc = acc + _shuffle(acc, perm)
    return acc


def _normalize_transpose(in2d, tp1):
    lanes = lax.iota(jnp.int32, LANES)
    cvecs = []
    for k in range(DIM // LANES):
        d = k * LANES + lanes
        cvecs.append(
            lax.shift_left(lax.shift_right_logical(d, 3), 10)
            + lax.shift_left(lax.bitwise_and(d, 7), 7)
        )

    def row(r, carry):
        vs = [in2d[r, pl.ds(k * LANES, LANES)] for k in range(DIM // LANES)]
        acc = vs[0] * vs[0]
        for v in vs[1:]:
            acc = acc + v * v
        y = _rsqrt(_hsum(acc))
        bb = jnp.full((LANES,), 0, jnp.int32) + r
        for k, v in enumerate(vs):
            plsc.store_scatter(tp1, [cvecs[k] + bb], v * y)
        return carry

    lax.fori_loop(0, CHUNK, row, 0)


def _make_lookup(l_seq, nblk):
    total_chunks = l_seq * nblk
    steps = total_chunks // NW
    mesh = plsc.VectorSubcoreMesh(core_axis_name="c", subcore_axis_name="s")

    @functools.partial(
        pl.kernel,
        mesh=mesh,
        compiler_params=pltpu.CompilerParams(
            needs_layout_passes=False,
            use_tc_tiling_on_sc=False,
            disable_bounds_checks=True,
        ),
        out_type=jax.ShapeDtypeStruct(
            (l_seq, DIM // 8, nblk, 8 * CHUNK), jnp.float32
        ),
        scratch_types=[
            pltpu.VMEM((steps, CHUNK), jnp.int32),
            pltpu.VMEM((NBUF, CHUNK, DIM), jnp.float32),
            pltpu.VMEM((NBUF, 8 * CHUNK * (DIM // 8)), jnp.float32),
            pltpu.SemaphoreType.DMA((NBUF,)),
            pltpu.SemaphoreType.DMA((NBUF,)),
        ],
    )
    def lookup(x_hbm, tbl_hbm, out_hbm, idx_v, in_v, tp_v, sem_g, sem_s):
        w = lax.axis_index("s") * NC + lax.axis_index("c")
        pltpu.sync_copy(x_hbm.at[pl.ds(w * steps, steps)], idx_v)

        def gather(s, b):
            return pltpu.make_async_copy(
                tbl_hbm.at[idx_v.at[s]], in_v.at[b], sem_g.at[b]
            )

        nshift = nblk.bit_length() - 1

        plane = 8 * CHUNK

        def out_dma(lq, j, b, i):
            return pltpu.make_async_copy(
                tp_v.at[b].at[pl.ds(i * plane, plane)],
                out_hbm.at[lq].at[i].at[j],
                sem_s.at[b],
            )

        def drain_out(b):
            for i in range(DIM // 8):
                pltpu.make_async_copy(
                    out_hbm.at[0].at[0].at[0],
                    tp_v.at[b].at[pl.ds(0, plane)],
                    sem_s.at[b],
                ).wait()

        for b in range(NBUF):
            gather(b, b).start()

        def round_(t, carry):
            for b in range(NBUF):
                s = t * NBUF + b
                c = w * steps + s
                lq = lax.shift_right_logical(c, nshift)
                j = lax.bitwise_and(c, nblk - 1)

                @pl.when(s >= NBUF)
                def _():
                    drain_out(b)

                gather(s, b).wait()
                _normalize_transpose(in_v.at[b], tp_v.at[b])
                for i in range(DIM // 8):
                    out_dma(lq, j, b, i).start()

                @pl.when(s + NBUF < steps)
                def _():
                    gather(s + NBUF, b).start()

            return carry

        lax.fori_loop(0, steps // NBUF, round_, 0)
        for b in range(NBUF):
            drain_out(b)

    return lookup


def kernel(x, W_inner):
    b, l = x.shape
    nblk = b // CHUNK
    xlin = jnp.transpose(x).astype(jnp.int32).reshape(l * b)
    xi = xlin.reshape((l * b) // CHUNK, CHUNK)
    wf = lax.optimization_barrier(W_inner.reshape(-1))
    tbl = wf.reshape(W_inner.shape)
    out4 = _make_lookup(l, nblk)(xi, tbl)
    out5 = out4.reshape(l, DIM // 8, nblk, 8, CHUNK)
    return out5.transpose((2, 4, 0, 1, 3)).reshape(b, l, DIM)


# speedup vs baseline: 1.8063x; 1.8063x over previous
import functools

import jax
import jax.numpy as jnp
from jax import lax
from jax.experimental import pallas as pl
from jax.experimental.pallas import tpu as pltpu
from jax.experimental.pallas import tpu_sc as plsc

DIM = 64
CHUNK = 128
LANES = 16
NC, NS = 2, 16
NW = NC * NS
GROUPS = CHUNK // LANES


def _rsqrt(s):
    i = plsc.bitcast(s, jnp.int32)
    i = jnp.int32(0x5F3759DF) - lax.shift_right_logical(i, 1)
    y = plsc.bitcast(i, jnp.float32)
    for _ in range(3):
        y = y * (1.5 - 0.5 * s * y * y)
    return y


def _shuffle(x, idx):
    dn = lax.GatherDimensionNumbers(
        offset_dims=(), collapsed_slice_dims=(0,), start_index_map=(0,)
    )
    return lax.gather(
        x, idx[:, None], dn, (1,),
        mode=lax.GatherScatterMode.PROMISE_IN_BOUNDS,
    )


def _hsum(acc):
    lanes = lax.iota(jnp.int32, LANES)
    for sh in (8, 4, 2, 1):
        perm = jnp.bitwise_xor(lanes, sh)
        acc = acc + _shuffle(acc, perm)
    return acc


def _normalize_row(in_v, out_v, r, carry):
    vs = [in_v[r, pl.ds(k * LANES, LANES)] for k in range(DIM // LANES)]
    acc = vs[0] * vs[0]
    for v in vs[1:]:
        acc = acc + v * v
    y = _rsqrt(_hsum(acc))
    for k, v in enumerate(vs):
        out_v[r, pl.ds(k * LANES, LANES)] = v * y
    return carry


NBUF = 4


def _make_lookup(n_rows):
    steps = n_rows // (NW * CHUNK)
    mesh = plsc.VectorSubcoreMesh(core_axis_name="c", subcore_axis_name="s")

    @functools.partial(
        pl.kernel,
        mesh=mesh,
        compiler_params=pltpu.CompilerParams(
            needs_layout_passes=False, use_tc_tiling_on_sc=False
        ),
        out_type=jax.ShapeDtypeStruct((n_rows, DIM), jnp.float32),
        scratch_types=[
            pltpu.VMEM((steps, CHUNK), jnp.int32),
            pltpu.VMEM((NBUF, CHUNK, DIM), jnp.float32),
            pltpu.VMEM((NBUF, CHUNK, DIM), jnp.float32),
            pltpu.SemaphoreType.DMA((NBUF,)),
            pltpu.SemaphoreType.DMA((NBUF,)),
        ],
    )
    def lookup(x_hbm, tbl_hbm, out_hbm, idx_v, in_v, out_v, sem_g, sem_s):
        w = lax.axis_index("s") * NC + lax.axis_index("c")
        pltpu.sync_copy(x_hbm.at[pl.ds(w * steps, steps)], idx_v)

        def gather(s, b):
            return pltpu.make_async_copy(
                tbl_hbm.at[idx_v.at[s]], in_v.at[b], sem_g.at[b]
            )

        def scatter(s, b):
            base = (w * steps + s) * CHUNK
            return pltpu.make_async_copy(
                out_v.at[b], out_hbm.at[pl.ds(base, CHUNK)], sem_s.at[b]
            )

        for b in range(NBUF):
            gather(b, b).start()

        def round_(t, carry):
            for b in range(NBUF):
                s = t * NBUF + b

                @pl.when(s >= NBUF)
                def _():
                    scatter(s - NBUF, b).wait()

                gather(s, b).wait()
                lax.fori_loop(
                    0,
                    CHUNK,
                    functools.partial(_normalize_row, in_v.at[b], out_v.at[b]),
                    0,
                )
                scatter(s, b).start()

                @pl.when(s + NBUF < steps)
                def _():
                    gather(s + NBUF, b).start()

            return carry

        lax.fori_loop(0, steps // NBUF, round_, 0)
        for b in range(NBUF):
            scatter(steps - NBUF + b, b).wait()

    return lookup


def kernel(x, W_inner):
    b, l = x.shape
    n = b * l
    xi = x.astype(jnp.int32).reshape(n // CHUNK, CHUNK)
    out = _make_lookup(n)(xi, W_inner)
    return out.reshape(b, l, DIM)
